# baseline (speedup 1.0000x reference)
import functools

import jax
import jax.numpy as jnp
from jax import lax
from jax.experimental import pallas as pl
from jax.experimental.pallas import tpu as pltpu
from jax.experimental.pallas import tpu_sc as plsc

N_NODES = 10000
N_EDGES = 320000
D = 128
NEG_W = 1.0 / 5.0

NC = 2
NS = 16
NW = NC * NS
L = 16

EDGES_PER_TILE = N_EDGES // NW
CHUNK = 48
NFULL = EDGES_PER_TILE // CHUNK
TAIL = EDGES_PER_TILE - NFULL * CHUNK
PB = D // 2
DP = 3 * PB


def _table_body(am_ref, ph_ref, t_ref):
    am = am_ref[...]
    ph = ph_ref[...]
    t_ref[:, 0:D] = am.astype(jnp.bfloat16)
    t_ref[:, D:2 * D] = (am * jnp.cos(ph)).astype(jnp.bfloat16)
    t_ref[:, 2 * D:3 * D] = (am * jnp.sin(ph)).astype(jnp.bfloat16)


def _build_table(am, ph):
    t = pl.pallas_call(
        _table_body,
        grid=(10,),
        in_specs=[pl.BlockSpec((N_NODES // 10, D), lambda i: (i, 0)),
                  pl.BlockSpec((N_NODES // 10, D), lambda i: (i, 0))],
        out_specs=pl.BlockSpec((N_NODES // 10, 3 * D), lambda i: (i, 0)),
        out_shape=jax.ShapeDtypeStruct((N_NODES, 3 * D), jnp.bfloat16),
    )(am, ph)
    return jax.lax.bitcast_convert_type(
        t.reshape(N_NODES, DP, 2), jnp.float32)


def _sc_body(t_hbm, idx_hbm, r_hbm, i_hbm, b_hbm,
             idx_c, rows0, rows1, res_r, res_i, res_b, sem0, sem1):
    wid = lax.axis_index("s") * NC + lax.axis_index("c")
    lane = lax.iota(jnp.int32, 16)

    pltpu.sync_copy(idx_hbm.at[pl.ds(wid * 2 * EDGES_PER_TILE,
                                     2 * EDGES_PER_TILE)], idx_c)

    def issue(c, m, buf, sem):
        pltpu.async_copy(t_hbm.at[idx_c.at[pl.ds(c * 2 * CHUNK, m)]], buf, sem)

    def wait(m, buf, sem):
        pltpu.make_async_copy(t_hbm.at[idx_c.at[pl.ds(0, m)]], buf, sem).wait()

    def compute(c, n, rows):
        base = c * CHUNK

        def k_body(k, accs):
            ca = lane + k
            ca = jnp.where(ca >= PB, ca - PB, ca)
            cp = ca + PB
            cq = ca + 2 * PB
            out = []
            for g in range(n // L):
                rs_id = lane + (g * L)
                rd_id = rs_id + n
                a_s = plsc.bitcast(plsc.load_gather(rows, [rs_id, ca]), jnp.bfloat16)
                a_d = plsc.bitcast(plsc.load_gather(rows, [rd_id, ca]), jnp.bfloat16)
                p_s = plsc.bitcast(plsc.load_gather(rows, [rs_id, cp]), jnp.bfloat16)
                p_d = plsc.bitcast(plsc.load_gather(rows, [rd_id, cp]), jnp.bfloat16)
                q_s = plsc.bitcast(plsc.load_gather(rows, [rs_id, cq]), jnp.bfloat16)
                q_d = plsc.bitcast(plsc.load_gather(rows, [rd_id, cq]), jnp.bfloat16)
                ar, ai, ab = accs[g]
                r0, r1 = plsc.unpack(p_s * p_d + q_s * q_d,
                                     format=plsc.PackFormat.INTERLEAVED)
                i0, i1 = plsc.unpack(q_s * p_d - p_s * q_d,
                                     format=plsc.PackFormat.INTERLEAVED)
                b0, b1 = plsc.unpack(a_s * a_d,
                                     format=plsc.PackFormat.INTERLEAVED)
                out.append((ar + r0 + r1, ai + i0 + i1, ab + b0 + b1))
            return tuple(out)

        zeros = jnp.zeros((L,), jnp.float32)
        init = tuple((zeros, zeros, zeros) for _ in range(n // L))
        accs = lax.fori_loop(0, PB, k_body, init, unroll=4)
        for g in range(n // L):
            ar, ai, ab = accs[g]
            res_r[pl.ds(base + g * L, L)] = ar
            res_i[pl.ds(base + g * L, L)] = ai
            res_b[pl.ds(base + g * L, L)] = ab

    issue(0, 2 * CHUNK, rows0, sem0)

    def pair_body(i, carry):
        c0 = 2 * i
        wait(2 * CHUNK, rows0, sem0)
        issue(c0 + 1, 2 * CHUNK, rows1, sem1)
        compute(c0, CHUNK, rows0)
        wait(2 * CHUNK, rows1, sem1)

        @pl.when(c0 + 2 < NFULL)
        def _():
            issue(c0 + 2, 2 * CHUNK, rows0, sem0)

        @pl.when(c0 + 2 == NFULL)
        def _():
            issue(NFULL, 2 * TAIL, rows0.at[pl.ds(0, 2 * TAIL)], sem0)

        compute(c0 + 1, CHUNK, rows1)
        return carry

    lax.fori_loop(0, NFULL // 2, pair_body, 0)
    wait(2 * TAIL, rows0.at[pl.ds(0, 2 * TAIL)], sem0)
    compute(NFULL, TAIL, rows0)

    tile_base = wid * EDGES_PER_TILE
    pltpu.sync_copy(res_r, r_hbm.at[pl.ds(tile_base, EDGES_PER_TILE)])
    pltpu.sync_copy(res_i, i_hbm.at[pl.ds(tile_base, EDGES_PER_TILE)])
    pltpu.sync_copy(res_b, b_hbm.at[pl.ds(tile_base, EDGES_PER_TILE)])


_sc_dots = functools.partial(
    pl.kernel,
    out_type=[jax.ShapeDtypeStruct((N_EDGES,), jnp.float32)] * 3,
    mesh=plsc.VectorSubcoreMesh(core_axis_name="c", subcore_axis_name="s"),
    compiler_params=pltpu.CompilerParams(use_tc_tiling_on_sc=False,
                                         needs_layout_passes=False),
    scratch_types=[
        pltpu.VMEM((2 * EDGES_PER_TILE,), jnp.int32),
        pltpu.VMEM((2 * CHUNK, DP), jnp.float32),
        pltpu.VMEM((2 * CHUNK, DP), jnp.float32),
        pltpu.VMEM((EDGES_PER_TILE,), jnp.float32),
        pltpu.VMEM((EDGES_PER_TILE,), jnp.float32),
        pltpu.VMEM((EDGES_PER_TILE,), jnp.float32),
        pltpu.SemaphoreType.DMA,
        pltpu.SemaphoreType.DMA,
    ],
)(_sc_body)


def _loss_body(r_ref, i_ref, b_ref, lab_ref, w_ref, out_ref):
    real = r_ref[...]
    img = i_ref[...]
    bi = b_ref[...]
    lab = lab_ref[...]
    ex_t = jnp.where(lab == 3, 0.0, 1.0)
    per = jnp.maximum(bi, 0.0) - bi * ex_t + jnp.log1p(jnp.exp(-jnp.abs(bi)))
    exist_loss = jnp.sum(per) * (1.0 / N_EDGES)

    p0 = -jnp.sqrt(real * real + (img + 1.0) ** 2)
    p1 = -jnp.sqrt(real * real + (img - 1.0) ** 2)
    p2 = -jnp.sqrt((real - 1.0) ** 2 + img * img)
    p3 = -jnp.sqrt(real * real + img * img)
    m = jnp.maximum(jnp.maximum(p0, p1), jnp.maximum(p2, p3))
    lse = m + jnp.log(jnp.exp(p0 - m) + jnp.exp(p1 - m)
                      + jnp.exp(p2 - m) + jnp.exp(p3 - m))
    plab = jnp.where(lab == 0, p0,
                     jnp.where(lab == 1, p1,
                               jnp.where(lab == 2, p2, p3)))
    wl = jnp.where(lab == 3, NEG_W, 1.0)
    ce = jnp.sum(wl * (lse - plab)) * (1.0 / N_EDGES)
    out_ref[0, 0] = ce + w_ref[0] * exist_loss


def _final_loss(r, i, b, lab, loss_weight):
    rows = N_EDGES // D
    out = pl.pallas_call(
        _loss_body,
        in_specs=[pl.BlockSpec(memory_space=pltpu.VMEM)] * 4
        + [pl.BlockSpec(memory_space=pltpu.SMEM)],
        out_specs=pl.BlockSpec(memory_space=pltpu.SMEM),
        out_shape=jax.ShapeDtypeStruct((1, 1), jnp.float32),
    )(r.reshape(rows, D), i.reshape(rows, D), b.reshape(rows, D),
      lab.reshape(rows, D), loss_weight.reshape(1))
    return out[0, 0]


def _interleave_idx(src, dst):
    sm = src.reshape(NW, EDGES_PER_TILE)
    dm = dst.reshape(NW, EDGES_PER_TILE)
    nh = NFULL * CHUNK
    head = jnp.concatenate(
        [sm[:, :nh].reshape(NW, NFULL, CHUNK),
         dm[:, :nh].reshape(NW, NFULL, CHUNK)], axis=2)
    tail = jnp.concatenate([sm[:, nh:], dm[:, nh:]], axis=1)
    return jnp.concatenate(
        [head.reshape(NW, 2 * nh), tail], axis=1).reshape(2 * N_EDGES)


def kernel(all_edges, am_outputs, ph_outputs, loss_weight):
    src = all_edges[:, 0]
    dst = all_edges[:, 1]
    lab = all_edges[:, 2]
    table = _build_table(am_outputs, ph_outputs)
    r, i, b = _sc_dots(table, _interleave_idx(src, dst))
    w = jnp.asarray(loss_weight, jnp.float32)
    return _final_loss(r, i, b, lab, w)


# speedup vs baseline: 1.6067x; 1.6067x over previous
import functools

import jax
import jax.numpy as jnp
from jax import lax
from jax.experimental import pallas as pl
from jax.experimental.pallas import tpu as pltpu
from jax.experimental.pallas import tpu_sc as plsc

N_NODES = 10000
N_EDGES = 320000
D = 128
NEG_W = 1.0 / 5.0

NC = 2
NS = 16
NW = NC * NS
L = 16

EDGES_PER_TILE = N_EDGES // NW
CHUNK = 48
NFULL = EDGES_PER_TILE // CHUNK
TAIL = EDGES_PER_TILE - NFULL * CHUNK
PB = D // 2
DP = 3 * PB


def _table_body(am_ref, ph_ref, t_ref):
    am = am_ref[...]
    ph = ph_ref[...]
    t_ref[:, 0:D] = am.astype(jnp.bfloat16)
    t_ref[:, D:2 * D] = (am * jnp.cos(ph)).astype(jnp.bfloat16)
    t_ref[:, 2 * D:3 * D] = (am * jnp.sin(ph)).astype(jnp.bfloat16)


def _build_table(am, ph):
    t = pl.pallas_call(
        _table_body,
        grid=(10,),
        in_specs=[pl.BlockSpec((N_NODES // 10, D), lambda i: (i, 0)),
                  pl.BlockSpec((N_NODES // 10, D), lambda i: (i, 0))],
        out_specs=pl.BlockSpec((N_NODES // 10, 3 * D), lambda i: (i, 0)),
        out_shape=jax.ShapeDtypeStruct((N_NODES, 3 * D), jnp.bfloat16),
    )(am, ph)
    return jax.lax.bitcast_convert_type(
        t.reshape(N_NODES, DP, 2), jnp.float32)


NBUF = 4


def _sc_body(t_hbm, idx_hbm, r_hbm, i_hbm, b_hbm,
             idx_c, bufs_s, bufs_d, res_r, res_i, res_b, sems_s, sems_d):
    wid = lax.axis_index("s") * NC + lax.axis_index("c")
    lane = lax.iota(jnp.int32, 16)

    pltpu.sync_copy(idx_hbm.at[pl.ds(wid * 2 * EDGES_PER_TILE,
                                     2 * EDGES_PER_TILE)], idx_c)

    def issue(c, n, b):
        o = c * 2 * CHUNK
        pltpu.async_copy(t_hbm.at[idx_c.at[pl.ds(o, n)]],
                         bufs_s[b].at[pl.ds(0, n)], sems_s[b])
        pltpu.async_copy(t_hbm.at[idx_c.at[pl.ds(o + n, n)]],
                         bufs_d[b].at[pl.ds(0, n)], sems_d[b])

    def wait(n, b):
        pltpu.make_async_copy(t_hbm.at[idx_c.at[pl.ds(0, n)]],
                              bufs_s[b].at[pl.ds(0, n)], sems_s[b]).wait()
        pltpu.make_async_copy(t_hbm.at[idx_c.at[pl.ds(0, n)]],
                              bufs_d[b].at[pl.ds(0, n)], sems_d[b]).wait()

    def compute(c, n, b):
        rs, rd = bufs_s[b], bufs_d[b]
        base = c * CHUNK

        def k_body(k, accs):
            ca = lane + k
            ca = jnp.where(ca >= PB, ca - PB, ca)
            cp = ca + PB
            cq = ca + 2 * PB
            out = []
            for g in range(n // L):
                rid = lane + (g * L)
                a_s = plsc.bitcast(plsc.load_gather(rs, [rid, ca]), jnp.bfloat16)
                a_d = plsc.bitcast(plsc.load_gather(rd, [rid, ca]), jnp.bfloat16)
                p_s = plsc.bitcast(plsc.load_gather(rs, [rid, cp]), jnp.bfloat16)
                p_d = plsc.bitcast(plsc.load_gather(rd, [rid, cp]), jnp.bfloat16)
                q_s = plsc.bitcast(plsc.load_gather(rs, [rid, cq]), jnp.bfloat16)
                q_d = plsc.bitcast(plsc.load_gather(rd, [rid, cq]), jnp.bfloat16)
                ar, ai, ab = accs[g]
                r0, r1 = plsc.unpack(p_s * p_d + q_s * q_d,
                                     format=plsc.PackFormat.INTERLEAVED)
                i0, i1 = plsc.unpack(q_s * p_d - p_s * q_d,
                                     format=plsc.PackFormat.INTERLEAVED)
                b0, b1 = plsc.unpack(a_s * a_d,
                                     format=plsc.PackFormat.INTERLEAVED)
                out.append((ar + r0 + r1, ai + i0 + i1, ab + b0 + b1))
            return tuple(out)

        zeros = jnp.zeros((L,), jnp.float32)
        init = tuple((zeros, zeros, zeros) for _ in range(n // L))
        accs = lax.fori_loop(0, PB, k_body, init, unroll=2)
        for g in range(n // L):
            ar, ai, ab = accs[g]
            res_r[pl.ds(base + g * L, L)] = ar
            res_i[pl.ds(base + g * L, L)] = ai
            res_b[pl.ds(base + g * L, L)] = ab

    for b in range(NBUF):
        issue(b, CHUNK, b)

    def ring_body(i, carry):
        c0 = NBUF * i
        for b in range(NBUF):
            c = c0 + b
            wait(CHUNK, b)
            nxt = c + NBUF

            @pl.when(nxt < NFULL)
            def _(nxt=nxt, b=b):
                issue(nxt, CHUNK, b)

            @pl.when(nxt == NFULL)
            def _(nxt=nxt, b=b):
                issue(NFULL, TAIL, b)

            compute(c, CHUNK, b)
        return carry

    lax.fori_loop(0, NFULL // NBUF, ring_body, 0)
    wait(TAIL, 0)
    compute(NFULL, TAIL, 0)

    tile_base = wid * EDGES_PER_TILE
    pltpu.sync_copy(res_r, r_hbm.at[pl.ds(tile_base, EDGES_PER_TILE)])
    pltpu.sync_copy(res_i, i_hbm.at[pl.ds(tile_base, EDGES_PER_TILE)])
    pltpu.sync_copy(res_b, b_hbm.at[pl.ds(tile_base, EDGES_PER_TILE)])


_sc_dots = functools.partial(
    pl.kernel,
    out_type=[jax.ShapeDtypeStruct((N_EDGES,), jnp.float32)] * 3,
    mesh=plsc.VectorSubcoreMesh(core_axis_name="c", subcore_axis_name="s"),
    compiler_params=pltpu.CompilerParams(use_tc_tiling_on_sc=False,
                                         needs_layout_passes=False),
    scratch_types=[
        pltpu.VMEM((2 * EDGES_PER_TILE,), jnp.int32),
        tuple(pltpu.VMEM((CHUNK, DP), jnp.float32) for _ in range(NBUF)),
        tuple(pltpu.VMEM((CHUNK, DP), jnp.float32) for _ in range(NBUF)),
        pltpu.VMEM((EDGES_PER_TILE,), jnp.float32),
        pltpu.VMEM((EDGES_PER_TILE,), jnp.float32),
        pltpu.VMEM((EDGES_PER_TILE,), jnp.float32),
        tuple(pltpu.SemaphoreType.DMA for _ in range(NBUF)),
        tuple(pltpu.SemaphoreType.DMA for _ in range(NBUF)),
    ],
)(_sc_body)


def _loss_body(r_ref, i_ref, b_ref, lab_ref, w_ref, out_ref):
    real = r_ref[...]
    img = i_ref[...]
    bi = b_ref[...]
    lab = lab_ref[...]
    ex_t = jnp.where(lab == 3, 0.0, 1.0)
    per = jnp.maximum(bi, 0.0) - bi * ex_t + jnp.log1p(jnp.exp(-jnp.abs(bi)))
    exist_loss = jnp.sum(per) * (1.0 / N_EDGES)

    p0 = -jnp.sqrt(real * real + (img + 1.0) ** 2)
    p1 = -jnp.sqrt(real * real + (img - 1.0) ** 2)
    p2 = -jnp.sqrt((real - 1.0) ** 2 + img * img)
    p3 = -jnp.sqrt(real * real + img * img)
    m = jnp.maximum(jnp.maximum(p0, p1), jnp.maximum(p2, p3))
    lse = m + jnp.log(jnp.exp(p0 - m) + jnp.exp(p1 - m)
                      + jnp.exp(p2 - m) + jnp.exp(p3 - m))
    plab = jnp.where(lab == 0, p0,
                     jnp.where(lab == 1, p1,
                               jnp.where(lab == 2, p2, p3)))
    wl = jnp.where(lab == 3, NEG_W, 1.0)
    ce = jnp.sum(wl * (lse - plab)) * (1.0 / N_EDGES)
    out_ref[0, 0] = ce + w_ref[0] * exist_loss


def _final_loss(r, i, b, lab, loss_weight):
    rows = N_EDGES // D
    out = pl.pallas_call(
        _loss_body,
        in_specs=[pl.BlockSpec(memory_space=pltpu.VMEM)] * 4
        + [pl.BlockSpec(memory_space=pltpu.SMEM)],
        out_specs=pl.BlockSpec(memory_space=pltpu.SMEM),
        out_shape=jax.ShapeDtypeStruct((1, 1), jnp.float32),
    )(r.reshape(rows, D), i.reshape(rows, D), b.reshape(rows, D),
      lab.reshape(rows, D), loss_weight.reshape(1))
    return out[0, 0]


def _interleave_idx(src, dst):
    sm = src.reshape(NW, EDGES_PER_TILE)
    dm = dst.reshape(NW, EDGES_PER_TILE)
    nh = NFULL * CHUNK
    head = jnp.concatenate(
        [sm[:, :nh].reshape(NW, NFULL, CHUNK),
         dm[:, :nh].reshape(NW, NFULL, CHUNK)], axis=2)
    tail = jnp.concatenate([sm[:, nh:], dm[:, nh:]], axis=1)
    return jnp.concatenate(
        [head.reshape(NW, 2 * nh), tail], axis=1).reshape(2 * N_EDGES)


def kernel(all_edges, am_outputs, ph_outputs, loss_weight):
    src = all_edges[:, 0]
    dst = all_edges[:, 1]
    lab = all_edges[:, 2]
    table = _build_table(am_outputs, ph_outputs)
    r, i, b = _sc_dots(table, _interleave_idx(src, dst))
    w = jnp.asarray(loss_weight, jnp.float32)
    return _final_loss(r, i, b, lab, w)
